# baseline (speedup 1.0000x reference)
import functools
import math

import jax
import jax.numpy as jnp
from jax import lax
from jax.experimental import pallas as pl
from jax.experimental.pallas import tpu as pltpu
from jax.experimental.pallas import tpu_sc as plsc

H = 128
LEVELS = ('symbol', 'term', 'literal', 'clause', 'proof')
INTER_LIST = (('s2t', 'symbol', 'term'), ('t2l', 'term', 'literal'),
              ('l2c', 'literal', 'clause'), ('c2p', 'clause', 'proof'))
NUM_LEVELS = 5
HEADS = 4
DH = H // HEADS
BN = 256
C = 128
W = 32


def _rup(a, b):
    return (a + b - 1) // b * b


@functools.lru_cache(None)
def _segsum_call(n_pad, e_pad):
    n_iter = e_pad // (W * C)
    n_grp = n_iter // 4
    rpt = n_pad // 16
    mesh = plsc.VectorSubcoreMesh(core_axis_name="c", subcore_axis_name="s")

    def body(tbl, src, dst, out,
             sv0, sv1, sv2, sv3, dv0, dv1, dv2, dv3,
             rows0, rows1, zbuf, acc,
             si0, si1, si2, si3, sg0, sg1, ss0, ss1):
        sv = (sv0, sv1, sv2, sv3)
        dv = (dv0, dv1, dv2, dv3)
        rows = (rows0, rows1)
        semi = (si0, si1, si2, si3)
        semg = (sg0, sg1)
        sems = (ss0, ss1)
        cid = lax.axis_index("c")
        sid = lax.axis_index("s")
        wid = sid * 2 + cid
        per_w = n_iter * C
        base0 = wid * per_w
        for i in range(8):
            for j in range(H // 16):
                zbuf[i, pl.ds(j * 16, 16)] = jnp.zeros((16,), jnp.float32)
        for j in range(2):
            pltpu.async_copy(src.at[pl.ds(base0 + j * C, C)], sv[j], semi[j])
            pltpu.async_copy(dst.at[pl.ds(base0 + j * C, C)], dv[j], semi[j])
        row0 = sid * rpt

        def zloop(r, carry):
            pltpu.sync_copy(zbuf, acc.at[pl.ds(row0 + r * 8, 8)])
            return carry
        lax.fori_loop(0, rpt // 8, zloop, 0)
        plsc.subcore_barrier()

        def grp(g, carry):
            i0 = g * 4
            for j in range(4):
                i = i0 + j
                b = j % 2
                q = j
                qn = (j + 2) % 4
                pltpu.make_async_copy(
                    src.at[pl.ds(base0 + i * C, C)], sv[q], semi[q]).wait()
                pltpu.make_async_copy(
                    dst.at[pl.ds(base0 + i * C, C)], dv[q], semi[q]).wait()
                if j >= 2:
                    pltpu.make_async_copy(
                        rows[b], acc.at[dv[qn]], sems[b]).wait()
                else:
                    @pl.when(g > 0)
                    def _():
                        pltpu.make_async_copy(
                            rows[b], acc.at[dv[qn]], sems[b]).wait()
                pltpu.async_copy(
                    src.at[pl.ds(base0 + (i + 2) * C, C)], sv[qn], semi[qn])
                pltpu.async_copy(
                    dst.at[pl.ds(base0 + (i + 2) * C, C)], dv[qn], semi[qn])
                pltpu.async_copy(tbl.at[sv[q]], rows[b], semg[b])
                pltpu.make_async_copy(tbl.at[sv[q]], rows[b], semg[b]).wait()
                pltpu.async_copy(rows[b], acc.at[dv[q]], sems[b], add=True)
            return carry
        lax.fori_loop(0, n_grp, grp, 0)
        pltpu.make_async_copy(
            src.at[pl.ds(base0 + n_iter * C, C)], sv[0], semi[0]).wait()
        pltpu.make_async_copy(
            dst.at[pl.ds(base0 + n_iter * C, C)], dv[0], semi[0]).wait()
        pltpu.make_async_copy(
            src.at[pl.ds(base0 + (n_iter + 1) * C, C)], sv[1], semi[1]).wait()
        pltpu.make_async_copy(
            dst.at[pl.ds(base0 + (n_iter + 1) * C, C)], dv[1], semi[1]).wait()
        pltpu.make_async_copy(rows[0], acc.at[dv[2]], sems[0]).wait()
        pltpu.make_async_copy(rows[1], acc.at[dv[3]], sems[1]).wait()
        plsc.subcore_barrier()
        pltpu.sync_copy(acc.at[pl.ds(row0, rpt)],
                        out.at[cid, pl.ds(row0, rpt)])

    return pl.kernel(
        body, mesh=mesh,
        out_type=jax.ShapeDtypeStruct((2, n_pad, H), jnp.float32),
        scratch_types=(
            [pltpu.VMEM((C,), jnp.int32) for _ in range(8)]
            + [pltpu.VMEM((C, H), jnp.float32) for _ in range(2)]
            + [pltpu.VMEM((8, H), jnp.float32),
               pltpu.VMEM_SHARED((n_pad, H), jnp.float32)]
            + [pltpu.SemaphoreType.DMA for _ in range(8)]))


def _count_partials(n_pad, e_pad, dst3):
    ones_tbl = jnp.ones((8, H), jnp.float32)
    zsrc = jnp.zeros(dst3.shape, jnp.int32)
    full = _segsum_call(n_pad, e_pad)(ones_tbl, zsrc, dst3)
    return full[:, :, :16]


def _dense_body(x_ref, p_ref, c_ref, ws_ref, wn_ref, b_ref, o_ref):
    cnt = jnp.maximum(c_ref[0, :, :1] + c_ref[1, :, :1], 1.0)
    m = (p_ref[0] + p_ref[1]) / cnt
    o_ref[...] = jnp.maximum(
        jnp.dot(x_ref[...], ws_ref[...], preferred_element_type=jnp.float32)
        + jnp.dot(m, wn_ref[...], preferred_element_type=jnp.float32)
        + b_ref[...], 0.0)


@functools.lru_cache(None)
def _dense_call(n_pad):
    g = n_pad // BN
    return pl.pallas_call(
        _dense_body,
        grid=(g,),
        in_specs=[
            pl.BlockSpec((BN, H), lambda i: (i, 0)),
            pl.BlockSpec((2, BN, H), lambda i: (0, i, 0)),
            pl.BlockSpec((2, BN, 16), lambda i: (0, i, 0)),
            pl.BlockSpec((H, H), lambda i: (0, 0)),
            pl.BlockSpec((H, H), lambda i: (0, 0)),
            pl.BlockSpec((1, H), lambda i: (0, 0)),
        ],
        out_specs=pl.BlockSpec((BN, H), lambda i: (i, 0)),
        out_shape=jax.ShapeDtypeStruct((n_pad, H), jnp.float32),
    )


def _inter_body(h_ref, p_ref, c_ref, w_ref, o_ref):
    cnt = jnp.maximum(c_ref[0, :, :1] + c_ref[1, :, :1], 1.0)
    agg = (p_ref[0] + p_ref[1]) / cnt
    o_ref[...] = jnp.maximum(
        h_ref[...]
        + jnp.dot(agg, w_ref[...], preferred_element_type=jnp.float32), 0.0)


@functools.lru_cache(None)
def _inter_call(n_pad):
    g = n_pad // BN
    return pl.pallas_call(
        _inter_body,
        grid=(g,),
        in_specs=[
            pl.BlockSpec((BN, H), lambda i: (i, 0)),
            pl.BlockSpec((2, BN, H), lambda i: (0, i, 0)),
            pl.BlockSpec((2, BN, 16), lambda i: (0, i, 0)),
            pl.BlockSpec((H, H), lambda i: (0, 0)),
        ],
        out_specs=pl.BlockSpec((BN, H), lambda i: (i, 0)),
        out_shape=jax.ShapeDtypeStruct((n_pad, H), jnp.float32),
    )


@functools.lru_cache(None)
def _mean_call(n_pad, n_real):
    g = n_pad // BN

    def body(x_ref, o_ref):
        i = pl.program_id(0)

        @pl.when(i == 0)
        def _():
            o_ref[...] = jnp.zeros_like(o_ref)

        rows = i * BN + lax.broadcasted_iota(jnp.int32, (BN, 1), 0)
        xm = jnp.where(rows < n_real, x_ref[...], 0.0)
        o_ref[...] += jnp.sum(xm, axis=0, keepdims=True) / n_real

    return pl.pallas_call(
        body,
        grid=(g,),
        in_specs=[pl.BlockSpec((BN, H), lambda i: (i, 0))],
        out_specs=pl.BlockSpec((1, H), lambda i: (0, 0)),
        out_shape=jax.ShapeDtypeStruct((1, H), jnp.float32),
    )


def _attn_body(h_ref, s_ref, wq_ref, wk_ref, wv_ref, wo_ref, o_ref):
    f32 = jnp.float32
    k5 = jnp.dot(s_ref[...], wk_ref[...], preferred_element_type=f32)
    v5 = jnp.dot(s_ref[...], wv_ref[...], preferred_element_type=f32)
    q = jnp.dot(h_ref[...], wq_ref[...], preferred_element_type=f32)
    hr = lax.broadcasted_iota(jnp.int32, (H, H), 0) // DH
    hc = lax.broadcasted_iota(jnp.int32, (H, H), 1) // DH
    bsum = (hr == hc).astype(f32)
    scale = 1.0 / math.sqrt(DH)
    logits = [jnp.dot(q * k5[l:l + 1, :], bsum, preferred_element_type=f32)
              * scale for l in range(NUM_LEVELS)]
    mx = logits[0]
    for l in range(1, NUM_LEVELS):
        mx = jnp.maximum(mx, logits[l])
    es = [jnp.exp(sl - mx) for sl in logits]
    den = es[0] + es[1] + es[2] + es[3] + es[4]
    ctx = sum(es[l] * v5[l:l + 1, :] for l in range(NUM_LEVELS)) / den
    o_ref[...] = h_ref[...] + jnp.dot(ctx, wo_ref[...],
                                      preferred_element_type=f32)


@functools.lru_cache(None)
def _attn_call(n_pad):
    g = n_pad // BN
    return pl.pallas_call(
        _attn_body,
        grid=(g,),
        in_specs=[
            pl.BlockSpec((BN, H), lambda i: (i, 0)),
            pl.BlockSpec((8, H), lambda i: (0, 0)),
            pl.BlockSpec((H, H), lambda i: (0, 0)),
            pl.BlockSpec((H, H), lambda i: (0, 0)),
            pl.BlockSpec((H, H), lambda i: (0, 0)),
            pl.BlockSpec((H, H), lambda i: (0, 0)),
        ],
        out_specs=pl.BlockSpec((BN, H), lambda i: (i, 0)),
        out_shape=jax.ShapeDtypeStruct((n_pad, H), jnp.float32),
    )


def _outp_body(h_ref, w_ref, b_ref, o_ref):
    o_ref[...] = jnp.dot(h_ref[...], w_ref[...],
                         preferred_element_type=jnp.float32) + b_ref[...]


@functools.lru_cache(None)
def _outp_call(n_pad, e_out):
    g = n_pad // BN
    return pl.pallas_call(
        _outp_body,
        grid=(g,),
        in_specs=[
            pl.BlockSpec((BN, H), lambda i: (i, 0)),
            pl.BlockSpec((H, e_out), lambda i: (0, 0)),
            pl.BlockSpec((1, e_out), lambda i: (0, 0)),
        ],
        out_specs=pl.BlockSpec((BN, e_out), lambda i: (i, 0)),
        out_shape=jax.ShapeDtypeStruct((n_pad, e_out), jnp.float32),
    )


def kernel(x_symbol, edge_symbol, W_self_symbol, W_nbr_symbol, b_symbol,
           x_term, edge_term, W_self_term, W_nbr_term, b_term,
           x_literal, edge_literal, W_self_literal, W_nbr_literal, b_literal,
           x_clause, edge_clause, W_self_clause, W_nbr_clause, b_clause,
           x_proof, edge_proof, W_self_proof, W_nbr_proof, b_proof,
           s2t_src, s2t_dst, W_inter_s2t,
           t2l_src, t2l_dst, W_inter_t2l,
           l2c_src, l2c_dst, W_inter_l2c,
           c2p_src, c2p_dst, W_inter_c2p,
           Wq, Wk, Wv, Wo, W_out, b_out):
    d = dict(locals())

    h, n_pad, n_real, ei, cnts = {}, {}, {}, {}, {}
    for lvl in LEVELS:
        n = d['x_' + lvl].shape[0]
        npd = _rup(n + 1, BN)
        n_real[lvl] = n
        n_pad[lvl] = npd
        h[lvl] = jnp.pad(d['x_' + lvl], ((0, npd - n), (0, 0)))
        e = d['edge_' + lvl].shape[1]
        ep = _rup(e, W * C * 4)
        src = jnp.pad(d['edge_' + lvl][0], (0, ep - e + 2 * C))
        dst = jnp.pad(d['edge_' + lvl][1], (0, ep - e + 2 * C),
                      constant_values=n)
        ei[lvl] = (src, dst, ep)
        cnts[lvl] = _count_partials(npd, ep, dst)
    for name, lo, hi in INTER_LIST:
        e = d[name + '_src'].shape[0]
        ep = _rup(e, W * C * 4)
        src = jnp.pad(d[name + '_src'], (0, ep - e + 2 * C))
        dst = jnp.pad(d[name + '_dst'], (0, ep - e + 2 * C),
                      constant_values=n_real[hi])
        ei[name] = (src, dst, ep)
        cnts[name] = _count_partials(n_pad[hi], ep, dst)

    for _rnd in range(2):
        for lvl in LEVELS:
            src, dst, ep = ei[lvl]
            for _l in range(3):
                part = _segsum_call(n_pad[lvl], ep)(h[lvl], src, dst)
                h[lvl] = _dense_call(n_pad[lvl])(
                    h[lvl], part, cnts[lvl], d['W_self_' + lvl],
                    d['W_nbr_' + lvl], d['b_' + lvl].reshape(1, H))
        for name, lo, hi in INTER_LIST:
            src, dst, ep = ei[name]
            part = _segsum_call(n_pad[hi], ep)(h[lo], src, dst)
            h[hi] = _inter_call(n_pad[hi])(
                h[hi], part, cnts[name], d['W_inter_' + name])
        summ = jnp.concatenate(
            [_mean_call(n_pad[lvl], n_real[lvl])(h[lvl]) for lvl in LEVELS]
            + [jnp.zeros((8 - NUM_LEVELS, H), jnp.float32)], axis=0)
        for lvl in LEVELS:
            h[lvl] = _attn_call(n_pad[lvl])(
                h[lvl], summ, d['Wq'], d['Wk'], d['Wv'], d['Wo'])

    e_out = W_out.shape[1]
    out = _outp_call(n_pad['clause'], e_out)(
        h['clause'], W_out, b_out.reshape(1, e_out))
    return out[:n_real['clause']]


# speedup vs baseline: 17.3973x; 17.3973x over previous
import functools
import math

import jax
import jax.numpy as jnp
from jax import lax
from jax.experimental import pallas as pl
from jax.experimental.pallas import tpu as pltpu
from jax.experimental.pallas import tpu_sc as plsc

H = 128
LEVELS = ('symbol', 'term', 'literal', 'clause', 'proof')
INTER_LIST = (('s2t', 'symbol', 'term'), ('t2l', 'term', 'literal'),
              ('l2c', 'literal', 'clause'), ('c2p', 'clause', 'proof'))
NUM_LEVELS = 5
HEADS = 4
DH = H // HEADS
BN = 256
C = 128
W = 32


def _rup(a, b):
    return (a + b - 1) // b * b


def _pad_edges(src, dst, pad, n_src, n_dst, n_dst_pad):
    ar = jnp.arange(pad, dtype=jnp.int32)
    src_p = jnp.concatenate([src, ar % min(1024, n_src)])
    dst_p = jnp.concatenate([dst, n_dst + ar % (n_dst_pad - n_dst)])
    return src_p, dst_p


@functools.lru_cache(None)
def _segsum_call(n_pad, e_pad):
    n_iter = e_pad // (W * C)
    n_grp = n_iter // 4
    rpt = n_pad // 16
    ZR = min(64, rpt)
    mesh = plsc.VectorSubcoreMesh(core_axis_name="c", subcore_axis_name="s")

    def body(tbl, src, dst, out,
             sv0, sv1, sv2, sv3, dv0, dv1, dv2, dv3,
             rows0, rows1, zbuf, acc,
             si0, si1, si2, si3, sg0, sg1, ss0, ss1):
        sv = (sv0, sv1, sv2, sv3)
        dv = (dv0, dv1, dv2, dv3)
        rows = (rows0, rows1)
        semi = (si0, si1, si2, si3)
        semg = (sg0, sg1)
        sems = (ss0, ss1)
        cid = lax.axis_index("c")
        sid = lax.axis_index("s")
        wid = sid * 2 + cid
        per_w = n_iter * C
        base0 = wid * per_w
        for i in range(ZR):
            for j in range(H // 16):
                zbuf[i, pl.ds(j * 16, 16)] = jnp.zeros((16,), jnp.float32)
        for j in range(2):
            pltpu.async_copy(src.at[pl.ds(base0 + j * C, C)], sv[j], semi[j])
            pltpu.async_copy(dst.at[pl.ds(base0 + j * C, C)], dv[j], semi[j])
        row0 = sid * rpt

        def zloop(r, carry):
            pltpu.sync_copy(zbuf, acc.at[pl.ds(row0 + r * ZR, ZR)])
            return carry
        lax.fori_loop(0, rpt // ZR, zloop, 0)
        plsc.subcore_barrier()

        def grp(g, carry):
            i0 = g * 4
            for j in range(4):
                i = i0 + j
                b = j % 2
                q = j
                qn = (j + 2) % 4
                pltpu.make_async_copy(
                    src.at[pl.ds(base0 + i * C, C)], sv[q], semi[q]).wait()
                pltpu.make_async_copy(
                    dst.at[pl.ds(base0 + i * C, C)], dv[q], semi[q]).wait()
                if j >= 2:
                    pltpu.make_async_copy(
                        rows[b], acc.at[dv[qn]], sems[b]).wait()
                else:
                    @pl.when(g > 0)
                    def _():
                        pltpu.make_async_copy(
                            rows[b], acc.at[dv[qn]], sems[b]).wait()
                pltpu.async_copy(
                    src.at[pl.ds(base0 + (i + 2) * C, C)], sv[qn], semi[qn])
                pltpu.async_copy(
                    dst.at[pl.ds(base0 + (i + 2) * C, C)], dv[qn], semi[qn])
                pltpu.async_copy(tbl.at[sv[q]], rows[b], semg[b])
                pltpu.make_async_copy(tbl.at[sv[q]], rows[b], semg[b]).wait()
                pltpu.async_copy(rows[b], acc.at[dv[q]], sems[b], add=True)
            return carry
        lax.fori_loop(0, n_grp, grp, 0)
        pltpu.make_async_copy(
            src.at[pl.ds(base0 + n_iter * C, C)], sv[0], semi[0]).wait()
        pltpu.make_async_copy(
            dst.at[pl.ds(base0 + n_iter * C, C)], dv[0], semi[0]).wait()
        pltpu.make_async_copy(
            src.at[pl.ds(base0 + (n_iter + 1) * C, C)], sv[1], semi[1]).wait()
        pltpu.make_async_copy(
            dst.at[pl.ds(base0 + (n_iter + 1) * C, C)], dv[1], semi[1]).wait()
        pltpu.make_async_copy(rows[0], acc.at[dv[2]], sems[0]).wait()
        pltpu.make_async_copy(rows[1], acc.at[dv[3]], sems[1]).wait()
        plsc.subcore_barrier()
        pltpu.sync_copy(acc.at[pl.ds(row0, rpt)],
                        out.at[cid, pl.ds(row0, rpt)])

    return pl.kernel(
        body, mesh=mesh,
        out_type=jax.ShapeDtypeStruct((2, n_pad, H), jnp.float32),
        scratch_types=(
            [pltpu.VMEM((C,), jnp.int32) for _ in range(8)]
            + [pltpu.VMEM((C, H), jnp.float32) for _ in range(2)]
            + [pltpu.VMEM((min(64, n_pad // 16), H), jnp.float32),
               pltpu.VMEM_SHARED((n_pad, H), jnp.float32)]
            + [pltpu.SemaphoreType.DMA for _ in range(8)]))


def _count_partials(n_pad, e_pad, dst3):
    ones_tbl = jnp.ones((1024, H), jnp.float32)
    zsrc = jnp.arange(dst3.shape[0], dtype=jnp.int32) % 1024
    full = _segsum_call(n_pad, e_pad)(ones_tbl, zsrc, dst3)
    return full[:, :, :16]


def _dense_body(x_ref, p_ref, c_ref, ws_ref, wn_ref, b_ref, o_ref):
    cnt = jnp.maximum(c_ref[0, :, :1] + c_ref[1, :, :1], 1.0)
    m = (p_ref[0] + p_ref[1]) / cnt
    o_ref[...] = jnp.maximum(
        jnp.dot(x_ref[...], ws_ref[...], preferred_element_type=jnp.float32)
        + jnp.dot(m, wn_ref[...], preferred_element_type=jnp.float32)
        + b_ref[...], 0.0)


@functools.lru_cache(None)
def _dense_call(n_pad):
    g = n_pad // BN
    return pl.pallas_call(
        _dense_body,
        grid=(g,),
        in_specs=[
            pl.BlockSpec((BN, H), lambda i: (i, 0)),
            pl.BlockSpec((2, BN, H), lambda i: (0, i, 0)),
            pl.BlockSpec((2, BN, 16), lambda i: (0, i, 0)),
            pl.BlockSpec((H, H), lambda i: (0, 0)),
            pl.BlockSpec((H, H), lambda i: (0, 0)),
            pl.BlockSpec((1, H), lambda i: (0, 0)),
        ],
        out_specs=pl.BlockSpec((BN, H), lambda i: (i, 0)),
        out_shape=jax.ShapeDtypeStruct((n_pad, H), jnp.float32),
    )


def _inter_body(h_ref, p_ref, c_ref, w_ref, o_ref):
    cnt = jnp.maximum(c_ref[0, :, :1] + c_ref[1, :, :1], 1.0)
    agg = (p_ref[0] + p_ref[1]) / cnt
    o_ref[...] = jnp.maximum(
        h_ref[...]
        + jnp.dot(agg, w_ref[...], preferred_element_type=jnp.float32), 0.0)


@functools.lru_cache(None)
def _inter_call(n_pad):
    g = n_pad // BN
    return pl.pallas_call(
        _inter_body,
        grid=(g,),
        in_specs=[
            pl.BlockSpec((BN, H), lambda i: (i, 0)),
            pl.BlockSpec((2, BN, H), lambda i: (0, i, 0)),
            pl.BlockSpec((2, BN, 16), lambda i: (0, i, 0)),
            pl.BlockSpec((H, H), lambda i: (0, 0)),
        ],
        out_specs=pl.BlockSpec((BN, H), lambda i: (i, 0)),
        out_shape=jax.ShapeDtypeStruct((n_pad, H), jnp.float32),
    )


@functools.lru_cache(None)
def _mean_call(n_pad, n_real):
    g = n_pad // BN

    def body(x_ref, o_ref):
        i = pl.program_id(0)

        @pl.when(i == 0)
        def _():
            o_ref[...] = jnp.zeros_like(o_ref)

        rows = i * BN + lax.broadcasted_iota(jnp.int32, (BN, 1), 0)
        xm = jnp.where(rows < n_real, x_ref[...], 0.0)
        o_ref[...] += jnp.sum(xm, axis=0, keepdims=True) / n_real

    return pl.pallas_call(
        body,
        grid=(g,),
        in_specs=[pl.BlockSpec((BN, H), lambda i: (i, 0))],
        out_specs=pl.BlockSpec((1, H), lambda i: (0, 0)),
        out_shape=jax.ShapeDtypeStruct((1, H), jnp.float32),
    )


def _attn_body(h_ref, s_ref, wq_ref, wk_ref, wv_ref, wo_ref, o_ref):
    f32 = jnp.float32
    k5 = jnp.dot(s_ref[...], wk_ref[...], preferred_element_type=f32)
    v5 = jnp.dot(s_ref[...], wv_ref[...], preferred_element_type=f32)
    q = jnp.dot(h_ref[...], wq_ref[...], preferred_element_type=f32)
    hr = lax.broadcasted_iota(jnp.int32, (H, H), 0) // DH
    hc = lax.broadcasted_iota(jnp.int32, (H, H), 1) // DH
    bsum = (hr == hc).astype(f32)
    scale = 1.0 / math.sqrt(DH)
    logits = [jnp.dot(q * k5[l:l + 1, :], bsum, preferred_element_type=f32)
              * scale for l in range(NUM_LEVELS)]
    mx = logits[0]
    for l in range(1, NUM_LEVELS):
        mx = jnp.maximum(mx, logits[l])
    es = [jnp.exp(sl - mx) for sl in logits]
    den = es[0] + es[1] + es[2] + es[3] + es[4]
    ctx = sum(es[l] * v5[l:l + 1, :] for l in range(NUM_LEVELS)) / den
    o_ref[...] = h_ref[...] + jnp.dot(ctx, wo_ref[...],
                                      preferred_element_type=f32)


@functools.lru_cache(None)
def _attn_call(n_pad):
    g = n_pad // BN
    return pl.pallas_call(
        _attn_body,
        grid=(g,),
        in_specs=[
            pl.BlockSpec((BN, H), lambda i: (i, 0)),
            pl.BlockSpec((8, H), lambda i: (0, 0)),
            pl.BlockSpec((H, H), lambda i: (0, 0)),
            pl.BlockSpec((H, H), lambda i: (0, 0)),
            pl.BlockSpec((H, H), lambda i: (0, 0)),
            pl.BlockSpec((H, H), lambda i: (0, 0)),
        ],
        out_specs=pl.BlockSpec((BN, H), lambda i: (i, 0)),
        out_shape=jax.ShapeDtypeStruct((n_pad, H), jnp.float32),
    )


def _outp_body(h_ref, w_ref, b_ref, o_ref):
    o_ref[...] = jnp.dot(h_ref[...], w_ref[...],
                         preferred_element_type=jnp.float32) + b_ref[...]


@functools.lru_cache(None)
def _outp_call(n_pad, e_out):
    g = n_pad // BN
    return pl.pallas_call(
        _outp_body,
        grid=(g,),
        in_specs=[
            pl.BlockSpec((BN, H), lambda i: (i, 0)),
            pl.BlockSpec((H, e_out), lambda i: (0, 0)),
            pl.BlockSpec((1, e_out), lambda i: (0, 0)),
        ],
        out_specs=pl.BlockSpec((BN, e_out), lambda i: (i, 0)),
        out_shape=jax.ShapeDtypeStruct((n_pad, e_out), jnp.float32),
    )


def kernel(x_symbol, edge_symbol, W_self_symbol, W_nbr_symbol, b_symbol,
           x_term, edge_term, W_self_term, W_nbr_term, b_term,
           x_literal, edge_literal, W_self_literal, W_nbr_literal, b_literal,
           x_clause, edge_clause, W_self_clause, W_nbr_clause, b_clause,
           x_proof, edge_proof, W_self_proof, W_nbr_proof, b_proof,
           s2t_src, s2t_dst, W_inter_s2t,
           t2l_src, t2l_dst, W_inter_t2l,
           l2c_src, l2c_dst, W_inter_l2c,
           c2p_src, c2p_dst, W_inter_c2p,
           Wq, Wk, Wv, Wo, W_out, b_out):
    d = dict(locals())

    h, n_pad, n_real, ei, cnts = {}, {}, {}, {}, {}
    for lvl in LEVELS:
        n = d['x_' + lvl].shape[0]
        npd = _rup(n + 1, BN)
        n_real[lvl] = n
        n_pad[lvl] = npd
        h[lvl] = jnp.pad(d['x_' + lvl], ((0, npd - n), (0, 0)))
        e = d['edge_' + lvl].shape[1]
        ep = _rup(e, W * C * 4)
        src, dst = _pad_edges(d['edge_' + lvl][0], d['edge_' + lvl][1],
                              ep - e + 2 * C, n, n, npd)
        ei[lvl] = (src, dst, ep)
        cnts[lvl] = _count_partials(npd, ep, dst)
    for name, lo, hi in INTER_LIST:
        e = d[name + '_src'].shape[0]
        ep = _rup(e, W * C * 4)
        src, dst = _pad_edges(d[name + '_src'], d[name + '_dst'],
                              ep - e + 2 * C, n_real[lo],
                              n_real[hi], n_pad[hi])
        ei[name] = (src, dst, ep)
        cnts[name] = _count_partials(n_pad[hi], ep, dst)

    for _rnd in range(2):
        for lvl in LEVELS:
            src, dst, ep = ei[lvl]
            for _l in range(3):
                part = _segsum_call(n_pad[lvl], ep)(h[lvl], src, dst)
                h[lvl] = _dense_call(n_pad[lvl])(
                    h[lvl], part, cnts[lvl], d['W_self_' + lvl],
                    d['W_nbr_' + lvl], d['b_' + lvl].reshape(1, H))
        for name, lo, hi in INTER_LIST:
            src, dst, ep = ei[name]
            part = _segsum_call(n_pad[hi], ep)(h[lo], src, dst)
            h[hi] = _inter_call(n_pad[hi])(
                h[hi], part, cnts[name], d['W_inter_' + name])
        summ = jnp.concatenate(
            [_mean_call(n_pad[lvl], n_real[lvl])(h[lvl]) for lvl in LEVELS]
            + [jnp.zeros((8 - NUM_LEVELS, H), jnp.float32)], axis=0)
        for lvl in LEVELS:
            h[lvl] = _attn_call(n_pad[lvl])(
                h[lvl], summ, d['Wq'], d['Wk'], d['Wv'], d['Wo'])

    e_out = W_out.shape[1]
    out = _outp_call(n_pad['clause'], e_out)(
        h['clause'], W_out, b_out.reshape(1, e_out))
    return out[:n_real['clause']]


# speedup vs baseline: 20.7230x; 1.1912x over previous
import functools
import math

import jax
import jax.numpy as jnp
from jax import lax
from jax.experimental import pallas as pl
from jax.experimental.pallas import tpu as pltpu
from jax.experimental.pallas import tpu_sc as plsc

H = 128
LEVELS = ('symbol', 'term', 'literal', 'clause', 'proof')
INTER_LIST = (('s2t', 'symbol', 'term'), ('t2l', 'term', 'literal'),
              ('l2c', 'literal', 'clause'), ('c2p', 'clause', 'proof'))
NUM_LEVELS = 5
HEADS = 4
DH = H // HEADS
BN = 256
C = 64
NR = 4
W = 32


def _rup(a, b):
    return (a + b - 1) // b * b


def _pad_edges(src, dst, pad, n_src, n_dst, n_dst_pad):
    ar = jnp.arange(pad, dtype=jnp.int32)
    src_p = jnp.concatenate([src, ar % min(1024, n_src)])
    dst_p = jnp.concatenate([dst, n_dst + ar % (n_dst_pad - n_dst)])
    return src_p, dst_p


@functools.lru_cache(None)
def _segsum_call(n_pad, e_pad):
    n_iter = e_pad // (W * C)
    n_grp2 = n_iter // (2 * NR)
    rpt = n_pad // 16
    ZR = min(64, rpt)
    NI = 2 * NR
    mesh = plsc.VectorSubcoreMesh(core_axis_name="c", subcore_axis_name="s")

    def body(tbl, src, dst, out, *refs):
        sv = refs[0:NI]
        dv = refs[NI:2 * NI]
        rows = refs[2 * NI:2 * NI + NR]
        zbuf = refs[2 * NI + NR]
        acc = refs[2 * NI + NR + 1]
        sems_base = 2 * NI + NR + 2
        semi = refs[sems_base:sems_base + NI]
        semg = refs[sems_base + NI:sems_base + NI + NR]
        sems = refs[sems_base + NI + NR:]
        cid = lax.axis_index("c")
        sid = lax.axis_index("s")
        wid = sid * 2 + cid
        per_w = n_iter * C
        base0 = wid * per_w
        for i in range(ZR):
            for j in range(H // 16):
                zbuf[i, pl.ds(j * 16, 16)] = jnp.zeros((16,), jnp.float32)
        for j in range(NR):
            pltpu.async_copy(src.at[pl.ds(base0 + j * C, C)], sv[j], semi[j])
            pltpu.async_copy(dst.at[pl.ds(base0 + j * C, C)], dv[j], semi[j])
        row0 = sid * rpt

        def zloop(r, carry):
            pltpu.sync_copy(zbuf, acc.at[pl.ds(row0 + r * ZR, ZR)])
            return carry
        lax.fori_loop(0, rpt // ZR, zloop, 0)
        plsc.subcore_barrier()

        def grp2(gp, carry):
            for half in range(2):
                g = gp * 2 + half
                i0v = g * NR
                for b in range(NR):
                    i = i0v + b
                    q = half * NR + b
                    qn = (1 - half) * NR + b
                    pltpu.make_async_copy(
                        src.at[pl.ds(base0 + i * C, C)], sv[q],
                        semi[q]).wait()
                    pltpu.make_async_copy(
                        dst.at[pl.ds(base0 + i * C, C)], dv[q],
                        semi[q]).wait()

                    @pl.when(g > 0)
                    def _():
                        pltpu.make_async_copy(
                            rows[b], acc.at[dv[qn]], sems[b]).wait()
                    pltpu.async_copy(
                        src.at[pl.ds(base0 + (i + NR) * C, C)], sv[qn],
                        semi[qn])
                    pltpu.async_copy(
                        dst.at[pl.ds(base0 + (i + NR) * C, C)], dv[qn],
                        semi[qn])
                    pltpu.async_copy(tbl.at[sv[q]], rows[b], semg[b])
                for b in range(NR):
                    q = half * NR + b
                    pltpu.make_async_copy(tbl.at[sv[q]], rows[b],
                                          semg[b]).wait()
                    pltpu.async_copy(rows[b], acc.at[dv[q]], sems[b],
                                     add=True)
            return carry
        lax.fori_loop(0, n_grp2, grp2, 0)
        for b in range(NR):
            i = n_iter + b
            pltpu.make_async_copy(
                src.at[pl.ds(base0 + i * C, C)], sv[b], semi[b]).wait()
            pltpu.make_async_copy(
                dst.at[pl.ds(base0 + i * C, C)], dv[b], semi[b]).wait()
        for b in range(NR):
            pltpu.make_async_copy(rows[b], acc.at[dv[NR + b]],
                                  sems[b]).wait()
        plsc.subcore_barrier()
        pltpu.sync_copy(acc.at[pl.ds(row0, rpt)],
                        out.at[cid, pl.ds(row0, rpt)])

    return pl.kernel(
        body, mesh=mesh,
        out_type=jax.ShapeDtypeStruct((2, n_pad, H), jnp.float32),
        scratch_types=(
            [pltpu.VMEM((C,), jnp.int32) for _ in range(2 * NI)]
            + [pltpu.VMEM((C, H), jnp.float32) for _ in range(NR)]
            + [pltpu.VMEM((min(64, n_pad // 16), H), jnp.float32),
               pltpu.VMEM_SHARED((n_pad, H), jnp.float32)]
            + [pltpu.SemaphoreType.DMA for _ in range(2 * NI + 2 * NR)]))


@functools.lru_cache(None)
def _count_call(n_pad, e_pad):
    CK = 128
    n_iter = e_pad // (W * CK)
    n_grp = n_iter // 4
    rpt = n_pad // 16
    ZR = min(64, rpt)
    mesh = plsc.VectorSubcoreMesh(core_axis_name="c", subcore_axis_name="s")

    def body(dst, out, dv0, dv1, dv2, dv3, ones_v, zbuf, acc,
             si0, si1, si2, si3, ss0, ss1):
        dv = (dv0, dv1, dv2, dv3)
        semi = (si0, si1, si2, si3)
        sems = (ss0, ss1)
        cid = lax.axis_index("c")
        sid = lax.axis_index("s")
        wid = sid * 2 + cid
        per_w = n_iter * CK
        base0 = wid * per_w
        for i in range(ZR):
            for j in range(H // 16):
                zbuf[i, pl.ds(j * 16, 16)] = jnp.zeros((16,), jnp.float32)
        for i in range(CK):
            for j in range(H // 16):
                ones_v[i, pl.ds(j * 16, 16)] = jnp.ones((16,), jnp.float32)
        for j in range(2):
            pltpu.async_copy(dst.at[pl.ds(base0 + j * CK, CK)], dv[j],
                             semi[j])
        row0 = sid * rpt

        def zloop(r, carry):
            pltpu.sync_copy(zbuf, acc.at[pl.ds(row0 + r * ZR, ZR)])
            return carry
        lax.fori_loop(0, rpt // ZR, zloop, 0)
        plsc.subcore_barrier()

        def grp(g, carry):
            i0 = g * 4
            for j in range(4):
                i = i0 + j
                b = j % 2
                q = j
                qn = (j + 2) % 4
                pltpu.make_async_copy(
                    dst.at[pl.ds(base0 + i * CK, CK)], dv[q], semi[q]).wait()
                if j >= 2:
                    pltpu.make_async_copy(
                        ones_v, acc.at[dv[qn]], sems[b]).wait()
                else:
                    @pl.when(g > 0)
                    def _():
                        pltpu.make_async_copy(
                            ones_v, acc.at[dv[qn]], sems[b]).wait()
                pltpu.async_copy(
                    dst.at[pl.ds(base0 + (i + 2) * CK, CK)], dv[qn],
                    semi[qn])
                pltpu.async_copy(ones_v, acc.at[dv[q]], sems[b], add=True)
            return carry
        lax.fori_loop(0, n_grp, grp, 0)
        pltpu.make_async_copy(
            dst.at[pl.ds(base0 + n_iter * CK, CK)], dv[0], semi[0]).wait()
        pltpu.make_async_copy(
            dst.at[pl.ds(base0 + (n_iter + 1) * CK, CK)], dv[1],
            semi[1]).wait()
        pltpu.make_async_copy(ones_v, acc.at[dv[2]], sems[0]).wait()
        pltpu.make_async_copy(ones_v, acc.at[dv[3]], sems[1]).wait()
        plsc.subcore_barrier()
        pltpu.sync_copy(acc.at[pl.ds(row0, rpt)],
                        out.at[cid, pl.ds(row0, rpt)])

    return pl.kernel(
        body, mesh=mesh,
        out_type=jax.ShapeDtypeStruct((2, n_pad, H), jnp.float32),
        scratch_types=(
            [pltpu.VMEM((128,), jnp.int32) for _ in range(4)]
            + [pltpu.VMEM((128, H), jnp.float32),
               pltpu.VMEM((min(64, n_pad // 16), H), jnp.float32),
               pltpu.VMEM_SHARED((n_pad, H), jnp.float32)]
            + [pltpu.SemaphoreType.DMA for _ in range(6)]))


def _count_partials(n_pad, e_pad, dst):
    full = _count_call(n_pad, e_pad)(dst)
    return full[:, :, :16]


def _dense_body(x_ref, p_ref, c_ref, ws_ref, wn_ref, b_ref, o_ref):
    cnt = jnp.maximum(c_ref[0, :, :1] + c_ref[1, :, :1], 1.0)
    m = (p_ref[0] + p_ref[1]) / cnt
    o_ref[...] = jnp.maximum(
        jnp.dot(x_ref[...], ws_ref[...], preferred_element_type=jnp.float32)
        + jnp.dot(m, wn_ref[...], preferred_element_type=jnp.float32)
        + b_ref[...], 0.0)


@functools.lru_cache(None)
def _dense_call(n_pad):
    g = n_pad // BN
    return pl.pallas_call(
        _dense_body,
        grid=(g,),
        in_specs=[
            pl.BlockSpec((BN, H), lambda i: (i, 0)),
            pl.BlockSpec((2, BN, H), lambda i: (0, i, 0)),
            pl.BlockSpec((2, BN, 16), lambda i: (0, i, 0)),
            pl.BlockSpec((H, H), lambda i: (0, 0)),
            pl.BlockSpec((H, H), lambda i: (0, 0)),
            pl.BlockSpec((1, H), lambda i: (0, 0)),
        ],
        out_specs=pl.BlockSpec((BN, H), lambda i: (i, 0)),
        out_shape=jax.ShapeDtypeStruct((n_pad, H), jnp.float32),
    )


def _inter_body(h_ref, p_ref, c_ref, w_ref, o_ref):
    cnt = jnp.maximum(c_ref[0, :, :1] + c_ref[1, :, :1], 1.0)
    agg = (p_ref[0] + p_ref[1]) / cnt
    o_ref[...] = jnp.maximum(
        h_ref[...]
        + jnp.dot(agg, w_ref[...], preferred_element_type=jnp.float32), 0.0)


@functools.lru_cache(None)
def _inter_call(n_pad):
    g = n_pad // BN
    return pl.pallas_call(
        _inter_body,
        grid=(g,),
        in_specs=[
            pl.BlockSpec((BN, H), lambda i: (i, 0)),
            pl.BlockSpec((2, BN, H), lambda i: (0, i, 0)),
            pl.BlockSpec((2, BN, 16), lambda i: (0, i, 0)),
            pl.BlockSpec((H, H), lambda i: (0, 0)),
        ],
        out_specs=pl.BlockSpec((BN, H), lambda i: (i, 0)),
        out_shape=jax.ShapeDtypeStruct((n_pad, H), jnp.float32),
    )


@functools.lru_cache(None)
def _mean_call(n_pad, n_real):
    g = n_pad // BN

    def body(x_ref, o_ref):
        i = pl.program_id(0)

        @pl.when(i == 0)
        def _():
            o_ref[...] = jnp.zeros_like(o_ref)

        rows = i * BN + lax.broadcasted_iota(jnp.int32, (BN, 1), 0)
        xm = jnp.where(rows < n_real, x_ref[...], 0.0)
        o_ref[...] += jnp.sum(xm, axis=0, keepdims=True) / n_real

    return pl.pallas_call(
        body,
        grid=(g,),
        in_specs=[pl.BlockSpec((BN, H), lambda i: (i, 0))],
        out_specs=pl.BlockSpec((1, H), lambda i: (0, 0)),
        out_shape=jax.ShapeDtypeStruct((1, H), jnp.float32),
    )


def _attn_body(h_ref, s_ref, wq_ref, wk_ref, wv_ref, wo_ref, o_ref):
    f32 = jnp.float32
    k5 = jnp.dot(s_ref[...], wk_ref[...], preferred_element_type=f32)
    v5 = jnp.dot(s_ref[...], wv_ref[...], preferred_element_type=f32)
    q = jnp.dot(h_ref[...], wq_ref[...], preferred_element_type=f32)
    hr = lax.broadcasted_iota(jnp.int32, (H, H), 0) // DH
    hc = lax.broadcasted_iota(jnp.int32, (H, H), 1) // DH
    bsum = (hr == hc).astype(f32)
    scale = 1.0 / math.sqrt(DH)
    logits = [jnp.dot(q * k5[l:l + 1, :], bsum, preferred_element_type=f32)
              * scale for l in range(NUM_LEVELS)]
    mx = logits[0]
    for l in range(1, NUM_LEVELS):
        mx = jnp.maximum(mx, logits[l])
    es = [jnp.exp(sl - mx) for sl in logits]
    den = es[0] + es[1] + es[2] + es[3] + es[4]
    ctx = sum(es[l] * v5[l:l + 1, :] for l in range(NUM_LEVELS)) / den
    o_ref[...] = h_ref[...] + jnp.dot(ctx, wo_ref[...],
                                      preferred_element_type=f32)


@functools.lru_cache(None)
def _attn_call(n_pad):
    g = n_pad // BN
    return pl.pallas_call(
        _attn_body,
        grid=(g,),
        in_specs=[
            pl.BlockSpec((BN, H), lambda i: (i, 0)),
            pl.BlockSpec((8, H), lambda i: (0, 0)),
            pl.BlockSpec((H, H), lambda i: (0, 0)),
            pl.BlockSpec((H, H), lambda i: (0, 0)),
            pl.BlockSpec((H, H), lambda i: (0, 0)),
            pl.BlockSpec((H, H), lambda i: (0, 0)),
        ],
        out_specs=pl.BlockSpec((BN, H), lambda i: (i, 0)),
        out_shape=jax.ShapeDtypeStruct((n_pad, H), jnp.float32),
    )


def _outp_body(h_ref, w_ref, b_ref, o_ref):
    o_ref[...] = jnp.dot(h_ref[...], w_ref[...],
                         preferred_element_type=jnp.float32) + b_ref[...]


@functools.lru_cache(None)
def _outp_call(n_pad, e_out):
    g = n_pad // BN
    return pl.pallas_call(
        _outp_body,
        grid=(g,),
        in_specs=[
            pl.BlockSpec((BN, H), lambda i: (i, 0)),
            pl.BlockSpec((H, e_out), lambda i: (0, 0)),
            pl.BlockSpec((1, e_out), lambda i: (0, 0)),
        ],
        out_specs=pl.BlockSpec((BN, e_out), lambda i: (i, 0)),
        out_shape=jax.ShapeDtypeStruct((n_pad, e_out), jnp.float32),
    )


def kernel(x_symbol, edge_symbol, W_self_symbol, W_nbr_symbol, b_symbol,
           x_term, edge_term, W_self_term, W_nbr_term, b_term,
           x_literal, edge_literal, W_self_literal, W_nbr_literal, b_literal,
           x_clause, edge_clause, W_self_clause, W_nbr_clause, b_clause,
           x_proof, edge_proof, W_self_proof, W_nbr_proof, b_proof,
           s2t_src, s2t_dst, W_inter_s2t,
           t2l_src, t2l_dst, W_inter_t2l,
           l2c_src, l2c_dst, W_inter_l2c,
           c2p_src, c2p_dst, W_inter_c2p,
           Wq, Wk, Wv, Wo, W_out, b_out):
    d = dict(locals())

    h, n_pad, n_real, ei, cnts = {}, {}, {}, {}, {}
    for lvl in LEVELS:
        n = d['x_' + lvl].shape[0]
        npd = _rup(n + 1, BN)
        n_real[lvl] = n
        n_pad[lvl] = npd
        h[lvl] = jnp.pad(d['x_' + lvl], ((0, npd - n), (0, 0)))
        e = d['edge_' + lvl].shape[1]
        ep = _rup(e, W * C * NR * 2)
        src, dst = _pad_edges(d['edge_' + lvl][0], d['edge_' + lvl][1],
                              ep - e + 2 * C * NR, n, n, npd)
        ei[lvl] = (src, dst, ep)
        cnts[lvl] = _count_partials(npd, ep, dst)
    for name, lo, hi in INTER_LIST:
        e = d[name + '_src'].shape[0]
        ep = _rup(e, W * C * NR * 2)
        src, dst = _pad_edges(d[name + '_src'], d[name + '_dst'],
                              ep - e + 2 * C * NR, n_real[lo],
                              n_real[hi], n_pad[hi])
        ei[name] = (src, dst, ep)
        cnts[name] = _count_partials(n_pad[hi], ep, dst)

    for _rnd in range(2):
        for lvl in LEVELS:
            src, dst, ep = ei[lvl]
            for _l in range(3):
                part = _segsum_call(n_pad[lvl], ep)(h[lvl], src, dst)
                h[lvl] = _dense_call(n_pad[lvl])(
                    h[lvl], part, cnts[lvl], d['W_self_' + lvl],
                    d['W_nbr_' + lvl], d['b_' + lvl].reshape(1, H))
        for name, lo, hi in INTER_LIST:
            src, dst, ep = ei[name]
            part = _segsum_call(n_pad[hi], ep)(h[lo], src, dst)
            h[hi] = _inter_call(n_pad[hi])(
                h[hi], part, cnts[name], d['W_inter_' + name])
        summ = jnp.concatenate(
            [_mean_call(n_pad[lvl], n_real[lvl])(h[lvl]) for lvl in LEVELS]
            + [jnp.zeros((8 - NUM_LEVELS, H), jnp.float32)], axis=0)
        for lvl in LEVELS:
            h[lvl] = _attn_call(n_pad[lvl])(
                h[lvl], summ, d['Wq'], d['Wk'], d['Wv'], d['Wo'])

    e_out = W_out.shape[1]
    out = _outp_call(n_pad['clause'], e_out)(
        h['clause'], W_out, b_out.reshape(1, e_out))
    return out[:n_real['clause']]


# speedup vs baseline: 20.9574x; 1.0113x over previous
import functools
import math

import jax
import jax.numpy as jnp
from jax import lax
from jax.experimental import pallas as pl
from jax.experimental.pallas import tpu as pltpu
from jax.experimental.pallas import tpu_sc as plsc

H = 128
LEVELS = ('symbol', 'term', 'literal', 'clause', 'proof')
INTER_LIST = (('s2t', 'symbol', 'term'), ('t2l', 'term', 'literal'),
              ('l2c', 'literal', 'clause'), ('c2p', 'clause', 'proof'))
NUM_LEVELS = 5
HEADS = 4
DH = H // HEADS
BN = 256
C = 64
NR = 4
W = 32


def _rup(a, b):
    return (a + b - 1) // b * b


def _pad_edges(src, dst, pad, n_src, n_dst, n_dst_pad):
    ar = jnp.arange(pad, dtype=jnp.int32)
    src_p = jnp.concatenate([src, ar % min(1024, n_src)])
    dst_p = jnp.concatenate([dst, n_dst + ar % (n_dst_pad - n_dst)])
    return src_p, dst_p


@functools.lru_cache(None)
def _segsum_call(n_pad, e_pad):
    n_iter = e_pad // (W * C)
    n_grp2 = n_iter // (2 * NR)
    rpt = n_pad // 16
    ZR = min(64, rpt)
    NI = 2 * NR
    mesh = plsc.VectorSubcoreMesh(core_axis_name="c", subcore_axis_name="s")

    def body(tbl, src, dst, out, *refs):
        sv = refs[0:NI]
        dv = refs[NI:2 * NI]
        rows = refs[2 * NI:2 * NI + NR]
        zbuf = refs[2 * NI + NR]
        acc = refs[2 * NI + NR + 1]
        sems_base = 2 * NI + NR + 2
        semi = refs[sems_base:sems_base + NI]
        semg = refs[sems_base + NI:sems_base + NI + NR]
        sems = refs[sems_base + NI + NR:]
        cid = lax.axis_index("c")
        sid = lax.axis_index("s")
        wid = sid * 2 + cid
        per_w = n_iter * C
        base0 = wid * per_w
        for i in range(ZR):
            for j in range(H // 16):
                zbuf[i, pl.ds(j * 16, 16)] = jnp.zeros((16,), jnp.float32)
        for j in range(NR):
            pltpu.async_copy(src.at[pl.ds(base0 + j * C, C)], sv[j], semi[j])
            pltpu.async_copy(dst.at[pl.ds(base0 + j * C, C)], dv[j], semi[j])
        row0 = sid * rpt
        for b in range(NR):
            pltpu.make_async_copy(
                src.at[pl.ds(base0 + b * C, C)], sv[b], semi[b]).wait()
            pltpu.make_async_copy(
                dst.at[pl.ds(base0 + b * C, C)], dv[b], semi[b]).wait()
            pltpu.async_copy(
                src.at[pl.ds(base0 + (b + NR) * C, C)], sv[NR + b],
                semi[NR + b])
            pltpu.async_copy(
                dst.at[pl.ds(base0 + (b + NR) * C, C)], dv[NR + b],
                semi[NR + b])
            pltpu.async_copy(tbl.at[sv[b]], rows[b], semg[b])

        def zloop(r, carry):
            pltpu.sync_copy(zbuf, acc.at[pl.ds(row0 + r * ZR, ZR)])
            return carry
        lax.fori_loop(0, rpt // ZR, zloop, 0)
        plsc.subcore_barrier()

        def grp2(gp, carry):
            for half in range(2):
                g = gp * 2 + half
                i0v = g * NR
                for b in range(NR):
                    i = i0v + b
                    q = half * NR + b
                    qn = (1 - half) * NR + b

                    @pl.when(g > 0)
                    def _():
                        pltpu.make_async_copy(
                            src.at[pl.ds(base0 + i * C, C)], sv[q],
                            semi[q]).wait()
                        pltpu.make_async_copy(
                            dst.at[pl.ds(base0 + i * C, C)], dv[q],
                            semi[q]).wait()
                        pltpu.make_async_copy(
                            rows[b], acc.at[dv[qn]], sems[b]).wait()
                        pltpu.async_copy(
                            src.at[pl.ds(base0 + (i + NR) * C, C)], sv[qn],
                            semi[qn])
                        pltpu.async_copy(
                            dst.at[pl.ds(base0 + (i + NR) * C, C)], dv[qn],
                            semi[qn])
                        pltpu.async_copy(tbl.at[sv[q]], rows[b], semg[b])
                for b in range(NR):
                    q = half * NR + b
                    pltpu.make_async_copy(tbl.at[sv[q]], rows[b],
                                          semg[b]).wait()
                    pltpu.async_copy(rows[b], acc.at[dv[q]], sems[b],
                                     add=True)
            return carry
        lax.fori_loop(0, n_grp2, grp2, 0)
        for b in range(NR):
            i = n_iter + b
            pltpu.make_async_copy(
                src.at[pl.ds(base0 + i * C, C)], sv[b], semi[b]).wait()
            pltpu.make_async_copy(
                dst.at[pl.ds(base0 + i * C, C)], dv[b], semi[b]).wait()
        for b in range(NR):
            pltpu.make_async_copy(rows[b], acc.at[dv[NR + b]],
                                  sems[b]).wait()
        plsc.subcore_barrier()
        pltpu.sync_copy(acc.at[pl.ds(row0, rpt)],
                        out.at[cid, pl.ds(row0, rpt)])

    return pl.kernel(
        body, mesh=mesh,
        out_type=jax.ShapeDtypeStruct((2, n_pad, H), jnp.float32),
        scratch_types=(
            [pltpu.VMEM((C,), jnp.int32) for _ in range(2 * NI)]
            + [pltpu.VMEM((C, H), jnp.float32) for _ in range(NR)]
            + [pltpu.VMEM((min(64, n_pad // 16), H), jnp.float32),
               pltpu.VMEM_SHARED((n_pad, H), jnp.float32)]
            + [pltpu.SemaphoreType.DMA for _ in range(2 * NI + 2 * NR)]))


@functools.lru_cache(None)
def _count_call(n_pad, e_pad):
    CK = 128
    n_iter = e_pad // (W * CK)
    n_grp = n_iter // 4
    rpt = n_pad // 16
    ZR = min(64, rpt)
    mesh = plsc.VectorSubcoreMesh(core_axis_name="c", subcore_axis_name="s")

    def body(dst, out, dv0, dv1, dv2, dv3, ones_v, zbuf, acc,
             si0, si1, si2, si3, ss0, ss1):
        dv = (dv0, dv1, dv2, dv3)
        semi = (si0, si1, si2, si3)
        sems = (ss0, ss1)
        cid = lax.axis_index("c")
        sid = lax.axis_index("s")
        wid = sid * 2 + cid
        per_w = n_iter * CK
        base0 = wid * per_w
        for i in range(ZR):
            for j in range(H // 16):
                zbuf[i, pl.ds(j * 16, 16)] = jnp.zeros((16,), jnp.float32)
        for i in range(CK):
            for j in range(H // 16):
                ones_v[i, pl.ds(j * 16, 16)] = jnp.ones((16,), jnp.float32)
        for j in range(2):
            pltpu.async_copy(dst.at[pl.ds(base0 + j * CK, CK)], dv[j],
                             semi[j])
        row0 = sid * rpt

        def zloop(r, carry):
            pltpu.sync_copy(zbuf, acc.at[pl.ds(row0 + r * ZR, ZR)])
            return carry
        lax.fori_loop(0, rpt // ZR, zloop, 0)
        plsc.subcore_barrier()

        def grp(g, carry):
            i0 = g * 4
            for j in range(4):
                i = i0 + j
                b = j % 2
                q = j
                qn = (j + 2) % 4
                pltpu.make_async_copy(
                    dst.at[pl.ds(base0 + i * CK, CK)], dv[q], semi[q]).wait()
                if j >= 2:
                    pltpu.make_async_copy(
                        ones_v, acc.at[dv[qn]], sems[b]).wait()
                else:
                    @pl.when(g > 0)
                    def _():
                        pltpu.make_async_copy(
                            ones_v, acc.at[dv[qn]], sems[b]).wait()
                pltpu.async_copy(
                    dst.at[pl.ds(base0 + (i + 2) * CK, CK)], dv[qn],
                    semi[qn])
                pltpu.async_copy(ones_v, acc.at[dv[q]], sems[b], add=True)
            return carry
        lax.fori_loop(0, n_grp, grp, 0)
        pltpu.make_async_copy(
            dst.at[pl.ds(base0 + n_iter * CK, CK)], dv[0], semi[0]).wait()
        pltpu.make_async_copy(
            dst.at[pl.ds(base0 + (n_iter + 1) * CK, CK)], dv[1],
            semi[1]).wait()
        pltpu.make_async_copy(ones_v, acc.at[dv[2]], sems[0]).wait()
        pltpu.make_async_copy(ones_v, acc.at[dv[3]], sems[1]).wait()
        plsc.subcore_barrier()
        pltpu.sync_copy(acc.at[pl.ds(row0, rpt)],
                        out.at[cid, pl.ds(row0, rpt)])

    return pl.kernel(
        body, mesh=mesh,
        out_type=jax.ShapeDtypeStruct((2, n_pad, H), jnp.float32),
        scratch_types=(
            [pltpu.VMEM((128,), jnp.int32) for _ in range(4)]
            + [pltpu.VMEM((128, H), jnp.float32),
               pltpu.VMEM((min(64, n_pad // 16), H), jnp.float32),
               pltpu.VMEM_SHARED((n_pad, H), jnp.float32)]
            + [pltpu.SemaphoreType.DMA for _ in range(6)]))


def _count_partials(n_pad, e_pad, dst):
    full = _count_call(n_pad, e_pad)(dst)
    return full[:, :, :16]


def _dense_body(x_ref, p_ref, c_ref, ws_ref, wn_ref, b_ref, o_ref):
    cnt = jnp.maximum(c_ref[0, :, :1] + c_ref[1, :, :1], 1.0)
    m = (p_ref[0] + p_ref[1]) / cnt
    o_ref[...] = jnp.maximum(
        jnp.dot(x_ref[...], ws_ref[...], preferred_element_type=jnp.float32)
        + jnp.dot(m, wn_ref[...], preferred_element_type=jnp.float32)
        + b_ref[...], 0.0)


@functools.lru_cache(None)
def _dense_call(n_pad):
    g = n_pad // BN
    return pl.pallas_call(
        _dense_body,
        grid=(g,),
        in_specs=[
            pl.BlockSpec((BN, H), lambda i: (i, 0)),
            pl.BlockSpec((2, BN, H), lambda i: (0, i, 0)),
            pl.BlockSpec((2, BN, 16), lambda i: (0, i, 0)),
            pl.BlockSpec((H, H), lambda i: (0, 0)),
            pl.BlockSpec((H, H), lambda i: (0, 0)),
            pl.BlockSpec((1, H), lambda i: (0, 0)),
        ],
        out_specs=pl.BlockSpec((BN, H), lambda i: (i, 0)),
        out_shape=jax.ShapeDtypeStruct((n_pad, H), jnp.float32),
    )


def _inter_body(h_ref, p_ref, c_ref, w_ref, o_ref):
    cnt = jnp.maximum(c_ref[0, :, :1] + c_ref[1, :, :1], 1.0)
    agg = (p_ref[0] + p_ref[1]) / cnt
    o_ref[...] = jnp.maximum(
        h_ref[...]
        + jnp.dot(agg, w_ref[...], preferred_element_type=jnp.float32), 0.0)


@functools.lru_cache(None)
def _inter_call(n_pad):
    g = n_pad // BN
    return pl.pallas_call(
        _inter_body,
        grid=(g,),
        in_specs=[
            pl.BlockSpec((BN, H), lambda i: (i, 0)),
            pl.BlockSpec((2, BN, H), lambda i: (0, i, 0)),
            pl.BlockSpec((2, BN, 16), lambda i: (0, i, 0)),
            pl.BlockSpec((H, H), lambda i: (0, 0)),
        ],
        out_specs=pl.BlockSpec((BN, H), lambda i: (i, 0)),
        out_shape=jax.ShapeDtypeStruct((n_pad, H), jnp.float32),
    )


@functools.lru_cache(None)
def _mean_call(n_pad, n_real):
    g = n_pad // BN

    def body(x_ref, o_ref):
        i = pl.program_id(0)

        @pl.when(i == 0)
        def _():
            o_ref[...] = jnp.zeros_like(o_ref)

        rows = i * BN + lax.broadcasted_iota(jnp.int32, (BN, 1), 0)
        xm = jnp.where(rows < n_real, x_ref[...], 0.0)
        o_ref[...] += jnp.sum(xm, axis=0, keepdims=True) / n_real

    return pl.pallas_call(
        body,
        grid=(g,),
        in_specs=[pl.BlockSpec((BN, H), lambda i: (i, 0))],
        out_specs=pl.BlockSpec((1, H), lambda i: (0, 0)),
        out_shape=jax.ShapeDtypeStruct((1, H), jnp.float32),
    )


def _attn_body(h_ref, s_ref, wq_ref, wk_ref, wv_ref, wo_ref, o_ref):
    f32 = jnp.float32
    k5 = jnp.dot(s_ref[...], wk_ref[...], preferred_element_type=f32)
    v5 = jnp.dot(s_ref[...], wv_ref[...], preferred_element_type=f32)
    q = jnp.dot(h_ref[...], wq_ref[...], preferred_element_type=f32)
    hr = lax.broadcasted_iota(jnp.int32, (H, H), 0) // DH
    hc = lax.broadcasted_iota(jnp.int32, (H, H), 1) // DH
    bsum = (hr == hc).astype(f32)
    scale = 1.0 / math.sqrt(DH)
    logits = [jnp.dot(q * k5[l:l + 1, :], bsum, preferred_element_type=f32)
              * scale for l in range(NUM_LEVELS)]
    mx = logits[0]
    for l in range(1, NUM_LEVELS):
        mx = jnp.maximum(mx, logits[l])
    es = [jnp.exp(sl - mx) for sl in logits]
    den = es[0] + es[1] + es[2] + es[3] + es[4]
    ctx = sum(es[l] * v5[l:l + 1, :] for l in range(NUM_LEVELS)) / den
    o_ref[...] = h_ref[...] + jnp.dot(ctx, wo_ref[...],
                                      preferred_element_type=f32)


@functools.lru_cache(None)
def _attn_call(n_pad):
    g = n_pad // BN
    return pl.pallas_call(
        _attn_body,
        grid=(g,),
        in_specs=[
            pl.BlockSpec((BN, H), lambda i: (i, 0)),
            pl.BlockSpec((8, H), lambda i: (0, 0)),
            pl.BlockSpec((H, H), lambda i: (0, 0)),
            pl.BlockSpec((H, H), lambda i: (0, 0)),
            pl.BlockSpec((H, H), lambda i: (0, 0)),
            pl.BlockSpec((H, H), lambda i: (0, 0)),
        ],
        out_specs=pl.BlockSpec((BN, H), lambda i: (i, 0)),
        out_shape=jax.ShapeDtypeStruct((n_pad, H), jnp.float32),
    )


def _outp_body(h_ref, w_ref, b_ref, o_ref):
    o_ref[...] = jnp.dot(h_ref[...], w_ref[...],
                         preferred_element_type=jnp.float32) + b_ref[...]


@functools.lru_cache(None)
def _outp_call(n_pad, e_out):
    g = n_pad // BN
    return pl.pallas_call(
        _outp_body,
        grid=(g,),
        in_specs=[
            pl.BlockSpec((BN, H), lambda i: (i, 0)),
            pl.BlockSpec((H, e_out), lambda i: (0, 0)),
            pl.BlockSpec((1, e_out), lambda i: (0, 0)),
        ],
        out_specs=pl.BlockSpec((BN, e_out), lambda i: (i, 0)),
        out_shape=jax.ShapeDtypeStruct((n_pad, e_out), jnp.float32),
    )


def kernel(x_symbol, edge_symbol, W_self_symbol, W_nbr_symbol, b_symbol,
           x_term, edge_term, W_self_term, W_nbr_term, b_term,
           x_literal, edge_literal, W_self_literal, W_nbr_literal, b_literal,
           x_clause, edge_clause, W_self_clause, W_nbr_clause, b_clause,
           x_proof, edge_proof, W_self_proof, W_nbr_proof, b_proof,
           s2t_src, s2t_dst, W_inter_s2t,
           t2l_src, t2l_dst, W_inter_t2l,
           l2c_src, l2c_dst, W_inter_l2c,
           c2p_src, c2p_dst, W_inter_c2p,
           Wq, Wk, Wv, Wo, W_out, b_out):
    d = dict(locals())

    h, n_pad, n_real, ei, cnts = {}, {}, {}, {}, {}
    for lvl in LEVELS:
        n = d['x_' + lvl].shape[0]
        npd = _rup(n + 1, BN)
        n_real[lvl] = n
        n_pad[lvl] = npd
        h[lvl] = jnp.pad(d['x_' + lvl], ((0, npd - n), (0, 0)))
        e = d['edge_' + lvl].shape[1]
        ep = _rup(e, W * C * NR * 2)
        src, dst = _pad_edges(d['edge_' + lvl][0], d['edge_' + lvl][1],
                              ep - e + 2 * C * NR, n, n, npd)
        ei[lvl] = (src, dst, ep)
        cnts[lvl] = _count_partials(npd, ep, dst)
    for name, lo, hi in INTER_LIST:
        e = d[name + '_src'].shape[0]
        ep = _rup(e, W * C * NR * 2)
        src, dst = _pad_edges(d[name + '_src'], d[name + '_dst'],
                              ep - e + 2 * C * NR, n_real[lo],
                              n_real[hi], n_pad[hi])
        ei[name] = (src, dst, ep)
        cnts[name] = _count_partials(n_pad[hi], ep, dst)

    for _rnd in range(2):
        for lvl in LEVELS:
            src, dst, ep = ei[lvl]
            for _l in range(3):
                part = _segsum_call(n_pad[lvl], ep)(h[lvl], src, dst)
                h[lvl] = _dense_call(n_pad[lvl])(
                    h[lvl], part, cnts[lvl], d['W_self_' + lvl],
                    d['W_nbr_' + lvl], d['b_' + lvl].reshape(1, H))
        for name, lo, hi in INTER_LIST:
            src, dst, ep = ei[name]
            part = _segsum_call(n_pad[hi], ep)(h[lo], src, dst)
            h[hi] = _inter_call(n_pad[hi])(
                h[hi], part, cnts[name], d['W_inter_' + name])
        summ = jnp.concatenate(
            [_mean_call(n_pad[lvl], n_real[lvl])(h[lvl]) for lvl in LEVELS]
            + [jnp.zeros((8 - NUM_LEVELS, H), jnp.float32)], axis=0)
        for lvl in LEVELS:
            h[lvl] = _attn_call(n_pad[lvl])(
                h[lvl], summ, d['Wq'], d['Wk'], d['Wv'], d['Wo'])

    e_out = W_out.shape[1]
    out = _outp_call(n_pad['clause'], e_out)(
        h['clause'], W_out, b_out.reshape(1, e_out))
    return out[:n_real['clause']]
